# baseline (speedup 1.0000x reference)
import functools

import jax
import jax.numpy as jnp
from jax import lax
from jax.experimental import pallas as pl
from jax.experimental.pallas import tpu as pltpu
from jax.experimental.pallas import tpu_sc as plsc

B = 4
N = 16384
NPOINT = 2048
NCORES = 2
NSUB = 16
WPB = 8
CHUNK = N // WPB
LANES = 16
NVREG = CHUNK // LANES
UNROLL = 2
BIG = 2**30
RECS = NSUB * LANES


def _fps_body(xs_hbm, ys_hbm, zs_hbm, out_hbm,
              xyz_ref, d_ref, out_buf, rec_ref, merge_ref, p0_ref, shared):
    c = lax.axis_index("c")
    s = lax.axis_index("s")
    lb = s // WPB
    k = s % WPB
    b = c * 2 + lb
    base = k * CHUNK

    pltpu.sync_copy(xs_hbm.at[b, pl.ds(base, CHUNK)],
                    xyz_ref.at[pl.ds(0, CHUNK)])
    pltpu.sync_copy(ys_hbm.at[b, pl.ds(base, CHUNK)],
                    xyz_ref.at[pl.ds(CHUNK, CHUNK)])
    pltpu.sync_copy(zs_hbm.at[b, pl.ds(base, CHUNK)],
                    xyz_ref.at[pl.ds(2 * CHUNK, CHUNK)])

    pltpu.sync_copy(xs_hbm.at[b, pl.ds(0, LANES)], p0_ref)
    p0x = p0_ref[...][0]
    pltpu.sync_copy(ys_hbm.at[b, pl.ds(0, LANES)], p0_ref)
    p0y = p0_ref[...][0]
    pltpu.sync_copy(zs_hbm.at[b, pl.ds(0, LANES)], p0_ref)
    p0z = p0_ref[...][0]

    init_v = jnp.full((LANES,), 1e10, jnp.float32)

    def init_body(j, _):
        d_ref[pl.ds(j * LANES, LANES)] = init_v
        return 0

    lax.fori_loop(0, NVREG, init_body, 0)

    lane = jax.lax.iota(jnp.int32, LANES)
    lane0 = lane == 0
    in_row = lane < WPB
    row_off = (lane & (WPB - 1)) * LANES
    win_pat = jnp.minimum(lane, 5)
    coord_pat = jnp.where(lane == 2, 0,
                jnp.where(lane == 3, CHUNK, 2 * CHUNK)).astype(jnp.int32)

    def scatter_scalar(ref, pos, val):
        plsc.store_scatter(ref, [jnp.full((LANES,), pos, jnp.int32)],
                           jnp.full((LANES,), val, ref.dtype), mask=lane0)

    @pl.when(k == 0)
    def _():
        scatter_scalar(out_buf, 0, jnp.int32(0))

    def outer(i, carry):
        lx, ly, lz = carry
        lxv = jnp.full((LANES,), lx, jnp.float32)
        lyv = jnp.full((LANES,), ly, jnp.float32)
        lzv = jnp.full((LANES,), lz, jnp.float32)

        vmax0 = jnp.full((LANES,), -1.0, jnp.float32)
        gbest0 = jnp.full((LANES,), BIG, jnp.int32)

        def scan(j, st):
            vmax, gbest = st
            xv = xyz_ref[pl.ds(j * LANES, LANES)]
            yv = xyz_ref[pl.ds(CHUNK + j * LANES, LANES)]
            zv = xyz_ref[pl.ds(2 * CHUNK + j * LANES, LANES)]
            dx = xv - lxv
            dy = yv - lyv
            dz = zv - lzv
            d = (dx * dx + dy * dy) + dz * dz
            sl = pl.ds(j * LANES, LANES)
            dn = jnp.minimum(d_ref[sl], d)
            d_ref[sl] = dn
            gcur = lane + (base + j * LANES)
            upd = (dn > vmax) | ((dn == vmax) & (gcur < gbest))
            vmax = jnp.where(upd, dn, vmax)
            gbest = jnp.where(upd, gcur, gbest)
            return vmax, gbest

        vmax, gbest = plsc.parallel_loop(
            0, NVREG, unroll=UNROLL, carry=(vmax0, gbest0))(scan)

        m = jnp.max(vmax)
        lidx = jnp.min(jnp.where(vmax == m, gbest, BIG))
        off = lidx - base
        coords = plsc.load_gather(
            xyz_ref, [coord_pat + jnp.full((LANES,), off, jnp.int32)])

        rec = jnp.where(lane == 0, m,
              jnp.where(lane == 1, lidx.astype(jnp.float32), coords))
        rec_ref[...] = rec

        p = (i & 1) * RECS
        pltpu.sync_copy(rec_ref, shared.at[pl.ds(p + s * LANES, LANES)])
        plsc.subcore_barrier()
        pltpu.sync_copy(shared.at[pl.ds(p + lb * (WPB * LANES), WPB * LANES)],
                        merge_ref)

        vals = plsc.load_gather(merge_ref, [row_off])
        mv = jnp.max(jnp.where(in_row, vals, -1.0))
        w = jnp.min(jnp.where(in_row & (vals == mv), row_off, BIG))
        win = plsc.load_gather(merge_ref,
                               [win_pat + jnp.full((LANES,), w, jnp.int32)])

        @pl.when(k == 0)
        def _():
            scatter_scalar(out_buf, i, win[1].astype(jnp.int32))

        return win[2], win[3], win[4]

    lax.fori_loop(1, NPOINT, outer, (p0x, p0y, p0z))

    @pl.when(k == 0)
    def _():
        pltpu.sync_copy(out_buf, out_hbm.at[b])


@jax.jit
def _fps(xs, ys, zs):
    mesh = plsc.VectorSubcoreMesh(core_axis_name="c", subcore_axis_name="s",
                                  num_cores=NCORES, num_subcores=NSUB)
    f = pl.kernel(
        _fps_body,
        out_type=jax.ShapeDtypeStruct((B, NPOINT), jnp.int32),
        mesh=mesh,
        compiler_params=pltpu.CompilerParams(needs_layout_passes=False),
        scratch_types=[
            pltpu.VMEM((3 * CHUNK,), jnp.float32),
            pltpu.VMEM((CHUNK,), jnp.float32),
            pltpu.VMEM((NPOINT,), jnp.int32),
            pltpu.VMEM((LANES,), jnp.float32),
            pltpu.VMEM((WPB * LANES,), jnp.float32),
            pltpu.VMEM((LANES,), jnp.float32),
            pltpu.VMEM_SHARED((2 * RECS,), jnp.float32),
        ],
    )
    return f(xs, ys, zs)


def kernel(points_xyz, features):
    del features
    xs = points_xyz[:, :, 0]
    ys = points_xyz[:, :, 1]
    zs = points_xyz[:, :, 2]
    return _fps(xs, ys, zs)


# speedup vs baseline: 1.0255x; 1.0255x over previous
import functools

import jax
import jax.numpy as jnp
from jax import lax
from jax.experimental import pallas as pl
from jax.experimental.pallas import tpu as pltpu
from jax.experimental.pallas import tpu_sc as plsc

B = 4
N = 16384
NPOINT = 2048
NCORES = 2
NSUB = 16
WPB = 8
CHUNK = N // WPB
LANES = 16
NVREG = CHUNK // LANES
UNROLL = 4
BIG = 2**30
RECS = NSUB * LANES


def _fps_body(xs_hbm, ys_hbm, zs_hbm, out_hbm,
              xyz_ref, d_ref, out_buf, rec_ref, merge_ref, p0_ref, shared):
    c = lax.axis_index("c")
    s = lax.axis_index("s")
    lb = s // WPB
    k = s % WPB
    b = c * 2 + lb
    base = k * CHUNK

    pltpu.sync_copy(xs_hbm.at[b, pl.ds(base, CHUNK)],
                    xyz_ref.at[pl.ds(0, CHUNK)])
    pltpu.sync_copy(ys_hbm.at[b, pl.ds(base, CHUNK)],
                    xyz_ref.at[pl.ds(CHUNK, CHUNK)])
    pltpu.sync_copy(zs_hbm.at[b, pl.ds(base, CHUNK)],
                    xyz_ref.at[pl.ds(2 * CHUNK, CHUNK)])

    pltpu.sync_copy(xs_hbm.at[b, pl.ds(0, LANES)], p0_ref)
    p0x = p0_ref[...][0]
    pltpu.sync_copy(ys_hbm.at[b, pl.ds(0, LANES)], p0_ref)
    p0y = p0_ref[...][0]
    pltpu.sync_copy(zs_hbm.at[b, pl.ds(0, LANES)], p0_ref)
    p0z = p0_ref[...][0]

    init_v = jnp.full((LANES,), 1e10, jnp.float32)

    def init_body(j, _):
        d_ref[pl.ds(j * LANES, LANES)] = init_v
        return 0

    lax.fori_loop(0, NVREG, init_body, 0)

    lane = jax.lax.iota(jnp.int32, LANES)
    lane0 = lane == 0
    in_row = lane < WPB
    row_off = (lane & (WPB - 1)) * LANES
    win_pat = jnp.minimum(lane, 5)
    coord_pat = jnp.where(lane == 2, 0,
                jnp.where(lane == 3, CHUNK, 2 * CHUNK)).astype(jnp.int32)

    def scatter_scalar(ref, pos, val):
        plsc.store_scatter(ref, [jnp.full((LANES,), pos, jnp.int32)],
                           jnp.full((LANES,), val, ref.dtype), mask=lane0)

    @pl.when(k == 0)
    def _():
        scatter_scalar(out_buf, 0, jnp.int32(0))

    def outer(i, carry):
        lx, ly, lz = carry
        lxv = jnp.full((LANES,), lx, jnp.float32)
        lyv = jnp.full((LANES,), ly, jnp.float32)
        lzv = jnp.full((LANES,), lz, jnp.float32)

        vmax0 = jnp.full((LANES,), -1.0, jnp.float32)
        gbest0 = jnp.full((LANES,), BIG, jnp.int32)

        def scan(j, st):
            vmax, gbest = st
            xv = xyz_ref[pl.ds(j * LANES, LANES)]
            yv = xyz_ref[pl.ds(CHUNK + j * LANES, LANES)]
            zv = xyz_ref[pl.ds(2 * CHUNK + j * LANES, LANES)]
            dx = xv - lxv
            dy = yv - lyv
            dz = zv - lzv
            d = (dx * dx + dy * dy) + dz * dz
            sl = pl.ds(j * LANES, LANES)
            dn = jnp.minimum(d_ref[sl], d)
            d_ref[sl] = dn
            gcur = lane + (base + j * LANES)
            upd = (dn > vmax) | ((dn == vmax) & (gcur < gbest))
            vmax = jnp.where(upd, dn, vmax)
            gbest = jnp.where(upd, gcur, gbest)
            return vmax, gbest

        vmax, gbest = plsc.parallel_loop(
            0, NVREG, unroll=UNROLL, carry=(vmax0, gbest0))(scan)

        m = jnp.max(vmax)
        lidx = jnp.min(jnp.where(vmax == m, gbest, BIG))
        off = lidx - base
        coords = plsc.load_gather(
            xyz_ref, [coord_pat + jnp.full((LANES,), off, jnp.int32)])

        rec = jnp.where(lane == 0, m,
              jnp.where(lane == 1, lidx.astype(jnp.float32), coords))
        rec_ref[...] = rec

        p = (i & 1) * RECS
        pltpu.sync_copy(rec_ref, shared.at[pl.ds(p + s * LANES, LANES)])
        plsc.subcore_barrier()
        pltpu.sync_copy(shared.at[pl.ds(p + lb * (WPB * LANES), WPB * LANES)],
                        merge_ref)

        vals = plsc.load_gather(merge_ref, [row_off])
        mv = jnp.max(jnp.where(in_row, vals, -1.0))
        w = jnp.min(jnp.where(in_row & (vals == mv), row_off, BIG))
        win = plsc.load_gather(merge_ref,
                               [win_pat + jnp.full((LANES,), w, jnp.int32)])

        @pl.when(k == 0)
        def _():
            scatter_scalar(out_buf, i, win[1].astype(jnp.int32))

        return win[2], win[3], win[4]

    lax.fori_loop(1, NPOINT, outer, (p0x, p0y, p0z))

    @pl.when(k == 0)
    def _():
        pltpu.sync_copy(out_buf, out_hbm.at[b])


@jax.jit
def _fps(xs, ys, zs):
    mesh = plsc.VectorSubcoreMesh(core_axis_name="c", subcore_axis_name="s",
                                  num_cores=NCORES, num_subcores=NSUB)
    f = pl.kernel(
        _fps_body,
        out_type=jax.ShapeDtypeStruct((B, NPOINT), jnp.int32),
        mesh=mesh,
        compiler_params=pltpu.CompilerParams(needs_layout_passes=False),
        scratch_types=[
            pltpu.VMEM((3 * CHUNK,), jnp.float32),
            pltpu.VMEM((CHUNK,), jnp.float32),
            pltpu.VMEM((NPOINT,), jnp.int32),
            pltpu.VMEM((LANES,), jnp.float32),
            pltpu.VMEM((WPB * LANES,), jnp.float32),
            pltpu.VMEM((LANES,), jnp.float32),
            pltpu.VMEM_SHARED((2 * RECS,), jnp.float32),
        ],
    )
    return f(xs, ys, zs)


def kernel(points_xyz, features):
    del features
    xs = points_xyz[:, :, 0]
    ys = points_xyz[:, :, 1]
    zs = points_xyz[:, :, 2]
    return _fps(xs, ys, zs)
